# Initial kernel scaffold; baseline (speedup 1.0000x reference)
import jax
import jax.numpy as jnp
from jax.experimental import pallas as pl


def kernel(x, edge_index, batch, W1, b1, W2, b2, W3, b3, W4, b4, g1, be1, g2, be2, g3, be3, g4, be4, Ws1, bs1, Ws2, bs2, Ha1, ha1, Hb1, hb1, Ha2, ha2, Hb2, hb2, Ha3, ha3, Hb3, hb3):
    raise NotImplementedError("write your pallas kernel here")


# speedup vs baseline: 1.0003x; 1.0003x over previous
import jax
import jax.numpy as jnp
from jax.experimental import pallas as pl

EPS = 1e-5
G = 128


def _gcn(x, W, b, src, dst, dis):
    h = x @ W
    n = x.shape[0]
    loop = jnp.arange(n, dtype=src.dtype)
    s = jnp.concatenate([src, loop])
    d = jnp.concatenate([dst, loop])
    norm = (dis[s] * dis[d])[:, None]
    out = jnp.zeros_like(h).at[d].add(h[s] * norm)
    return out + b


def _bn(x, g, b):
    m = jnp.mean(x, axis=0)
    v = jnp.var(x, axis=0)
    return (x - m) * jax.lax.rsqrt(v + EPS) * g + b


def _identity_kernel(x_ref, o_ref):
    o_ref[...] = x_ref[...]


def kernel(x, edge_index, batch, W1, b1, W2, b2, W3, b3, W4, b4, g1, be1, g2, be2, g3, be3, g4, be4, Ws1, bs1, Ws2, bs2, Ha1, ha1, Hb1, hb1, Ha2, ha2, Hb2, hb2, Ha3, ha3, Hb3, hb3):
    src = edge_index[0]
    dst = edge_index[1]
    n = x.shape[0]
    deg = jnp.zeros((n,), jnp.float32).at[dst].add(1.0) + 1.0
    dis = jax.lax.rsqrt(deg)
    h = jax.nn.relu(_bn(_gcn(x, W1, b1, src, dst, dis), g1, be1))
    h = jax.nn.relu(_bn(_gcn(h, W2, b2, src, dst, dis), g2, be2))
    h = jax.nn.relu(_bn(_gcn(h, W3, b3, src, dst, dis), g3, be3))
    h = jax.nn.relu(_bn(_gcn(h, W4, b4, src, dst, dis), g4, be4))
    cnt = jax.ops.segment_sum(jnp.ones((n,), jnp.float32), batch, num_segments=G)
    sm = jax.ops.segment_sum(h, batch, num_segments=G)
    mean = sm / jnp.maximum(cnt, 1.0)[:, None]
    mx = jax.ops.segment_max(h, batch, num_segments=G)
    mx = jnp.where(cnt[:, None] > 0, mx, 0.0)
    p = jnp.concatenate([mean, mx], axis=1)
    p = pl.pallas_call(
        _identity_kernel,
        out_shape=jax.ShapeDtypeStruct(p.shape, p.dtype),
    )(p)
    s = jax.nn.relu(p @ Ws1 + bs1)
    s = jax.nn.relu(s @ Ws2 + bs2)
    outs = []
    for Ha, ha, Hb, hb in ((Ha1, ha1, Hb1, hb1), (Ha2, ha2, Hb2, hb2), (Ha3, ha3, Hb3, hb3)):
        t = jax.nn.relu(s @ Ha + ha)
        outs.append((t @ Hb + hb).squeeze(-1))
    return tuple(outs)


# speedup vs baseline: 2.1260x; 2.1254x over previous
import functools

import jax
import jax.numpy as jnp
from jax import lax
from jax.experimental import pallas as pl
from jax.experimental.pallas import tpu as pltpu
from jax.experimental.pallas import tpu_sc as plsc

N = 50000
E = 800000
H = 256
G = 128
EPS = 1e-5
ECAT = E + N
WIN = 112
CH = 128
LPAD = ECAT + 2 * CH


def _tile_ranges(e_tot):
    n_win = -(-e_tot // WIN)
    base, extra = divmod(n_win, 16)
    ends, pos = [], 0
    for t in range(16):
        pos += (base + (1 if t < extra else 0)) * WIN
        ends.append(min(pos, e_tot))
    return list(zip([0] + ends[:-1], ends))


_R16 = _tile_ranges(ECAT)

_mesh = plsc.VectorSubcoreMesh(core_axis_name="c", subcore_axis_name="s")


@functools.partial(
    pl.kernel,
    out_type=(
        jax.ShapeDtypeStruct((N, H), jnp.float32),
        jax.ShapeDtypeStruct((32, H), jnp.float32),
    ),
    mesh=_mesh,
    scratch_types=[
        pltpu.VMEM((CH,), jnp.int32),
        pltpu.VMEM((CH + 16,), jnp.int32),
        pltpu.VMEM((CH + 16,), jnp.float32),
        pltpu.VMEM((CH, H), jnp.float32),
        pltpu.VMEM((H,), jnp.float32),
        pltpu.VMEM((112,), jnp.int32),
        pltpu.SemaphoreType.DMA,
    ],
)
def _sc_scatter(h_hbm, ssrc_hbm, sdst_hbm, snorm_hbm, se_hbm,
                out_hbm, headp_hbm,
                idx_v, dst_v, nrm_v, rowbuf, accbuf, se_v, sem):
    cid = lax.axis_index("c")
    sid = lax.axis_index("s")
    wid = sid * 2 + cid

    pltpu.sync_copy(se_hbm, se_v)
    start = se_v[pl.ds(wid, 16)][0]
    end = se_v[pl.ds(32 + wid, 16)][0]
    ismid = se_v[pl.ds(64 + wid, 16)][0]

    z16 = jnp.zeros((16,), jnp.float32)
    for k in range(16):
        accbuf[pl.ds(16 * k, 16)] = z16
    pltpu.sync_copy(accbuf, headp_hbm.at[wid])

    @pl.when(end > start)
    def _():
        abase = jnp.bitwise_and(start, jnp.int32(-16))
        nch = (end - abase + (CH - 1)) // CH

        def flush(cur, first, acc):
            for k in range(16):
                accbuf[pl.ds(16 * k, 16)] = acc[k]

            def to_head():
                pltpu.sync_copy(accbuf, headp_hbm.at[wid])

            def to_out():
                pltpu.sync_copy(accbuf, out_hbm.at[cur])

            lax.cond(jnp.logical_and(first == 1, ismid == 1), to_head, to_out)

        def chunk_body(ci, carry):
            cbase = pl.multiple_of(abase + ci * CH, 16)
            pltpu.sync_copy(ssrc_hbm.at[pl.ds(cbase, CH)], idx_v)
            pltpu.sync_copy(sdst_hbm.at[pl.ds(cbase, CH)], dst_v.at[pl.ds(0, CH)])
            pltpu.sync_copy(snorm_hbm.at[pl.ds(cbase, CH)], nrm_v.at[pl.ds(0, CH)])
            pltpu.async_copy(h_hbm.at[idx_v], rowbuf, sem).wait()
            lo = jnp.maximum(start - cbase, 0)
            hi = jnp.minimum(end - cbase, CH)

            def edge_body(j, ec):
                cur = ec[0]
                first = ec[1]
                acc = ec[2]
                d = dst_v[pl.ds(j, 16)][0]
                nv = nrm_v[pl.ds(j, 16)][0]
                changed = jnp.logical_and(d != cur, cur >= 0)
                lax.cond(changed, lambda: flush(cur, first, acc), lambda: None)
                first = jnp.where(changed, 0, first)
                acc = tuple(jnp.where(changed, z16, acc[k]) for k in range(16))
                acc = tuple(acc[k] + rowbuf[j, pl.ds(16 * k, 16)] * nv
                            for k in range(16))
                return (d, first, acc)

            return lax.fori_loop(lo, hi, edge_body, carry)

        carry0 = (jnp.int32(-1), jnp.int32(1), tuple(z16 for _ in range(16)))
        cur, first, acc = lax.fori_loop(0, nch, chunk_body, carry0)
        flush(cur, first, acc)


def _aggregate(h, ssrc_p, sdst_p, snorm_p, se, headrow):
    out, headp = _sc_scatter(h, ssrc_p, sdst_p, snorm_p, se)
    return out.at[headrow].add(headp)


def _bn(x, g, b):
    m = jnp.mean(x, axis=0)
    v = jnp.var(x, axis=0)
    return (x - m) * jax.lax.rsqrt(v + EPS) * g + b


def _pools(x, batch):
    cnt = jax.ops.segment_sum(jnp.ones((x.shape[0],), x.dtype), batch, num_segments=G)
    sm = jax.ops.segment_sum(x, batch, num_segments=G)
    mean = sm / jnp.maximum(cnt, 1.0)[:, None]
    mx = jax.ops.segment_max(x, batch, num_segments=G)
    mx = jnp.where(cnt[:, None] > 0, mx, 0.0)
    return jnp.concatenate([mean, mx], axis=1)


def kernel(x, edge_index, batch, W1, b1, W2, b2, W3, b3, W4, b4, g1, be1, g2, be2, g3, be3, g4, be4, Ws1, bs1, Ws2, bs2, Ha1, ha1, Hb1, hb1, Ha2, ha2, Hb2, hb2, Ha3, ha3, Hb3, hb3):
    src = edge_index[0]
    dst = edge_index[1]
    loop = jnp.arange(N, dtype=src.dtype)
    scat = jnp.concatenate([src, loop])
    dcat = jnp.concatenate([dst, loop])
    deg = jnp.zeros((N,), jnp.float32).at[dcat].add(1.0)
    dis = jax.lax.rsqrt(deg)
    norm = dis[scat] * dis[dcat]

    perm = jnp.argsort(dcat, stable=True)
    ssrc = scat[perm]
    sdst = dcat[perm]
    snorm = norm[perm]

    pad_i = jnp.zeros((LPAD - ECAT,), jnp.int32)
    ssrc_p = jnp.concatenate([ssrc, pad_i])
    sdst_p = jnp.concatenate([sdst, pad_i])
    snorm_p = jnp.concatenate([snorm, jnp.zeros((LPAD - ECAT,), jnp.float32)])

    starts, ends = [], []
    for (s0, e0) in _R16:
        m = (s0 + e0) // 2
        cutv = jnp.searchsorted(sdst, sdst[m], side='left').astype(jnp.int32)
        cut = jnp.clip(cutv, s0, e0)
        starts += [jnp.int32(s0), cut]
        ends += [cut, jnp.int32(e0)]
    starts = jnp.stack(starts)
    ends = jnp.stack(ends)
    prev = jnp.where(starts > 0, starts - 1, 0)
    is_mid = (starts > 0) & (sdst_p[starts] == sdst_p[prev]) & (ends > starts)
    se = jnp.concatenate([starts, ends, is_mid.astype(jnp.int32),
                          jnp.zeros((16,), jnp.int32)])
    headrow = jnp.where(is_mid, sdst_p[starts], -1)

    def gcn(h_in, W, b):
        h = h_in @ W
        out = _aggregate(h, ssrc_p, sdst_p, snorm_p, se, headrow)
        return out + b

    h = jax.nn.relu(_bn(gcn(x, W1, b1), g1, be1))
    h = jax.nn.relu(_bn(gcn(h, W2, b2), g2, be2))
    h = jax.nn.relu(_bn(gcn(h, W3, b3), g3, be3))
    h = jax.nn.relu(_bn(gcn(h, W4, b4), g4, be4))

    p = _pools(h, batch)
    s = jax.nn.relu(p @ Ws1 + bs1)
    s = jax.nn.relu(s @ Ws2 + bs2)
    outs = []
    for Ha, ha, Hb, hb in ((Ha1, ha1, Hb1, hb1), (Ha2, ha2, Hb2, hb2), (Ha3, ha3, Hb3, hb3)):
        t = jax.nn.relu(s @ Ha + ha)
        outs.append((t @ Hb + hb).squeeze(-1))
    return tuple(outs)


# speedup vs baseline: 4.5138x; 2.1232x over previous
import functools

import jax
import jax.numpy as jnp
from jax import lax
from jax.experimental import pallas as pl
from jax.experimental.pallas import tpu as pltpu
from jax.experimental.pallas import tpu_sc as plsc

N = 50000
E = 800000
H = 256
G = 128
EPS = 1e-5
ECAT = E + N
WIN = 112
CH = 128
SCH = 1024
SPER = 26 * SCH
LPAD = 32 * SPER


def _tile_ranges(e_tot):
    n_win = -(-e_tot // WIN)
    base, extra = divmod(n_win, 16)
    ends, pos = [], 0
    for t in range(16):
        pos += (base + (1 if t < extra else 0)) * WIN
        ends.append(min(pos, e_tot))
    return list(zip([0] + ends[:-1], ends))


_R16 = _tile_ranges(ECAT)

_mesh = plsc.VectorSubcoreMesh(core_axis_name="c", subcore_axis_name="s")


@functools.partial(
    pl.kernel,
    out_type=(
        jax.ShapeDtypeStruct((N, H), jnp.float32),
        jax.ShapeDtypeStruct((32, H), jnp.float32),
    ),
    mesh=_mesh,
    scratch_types=[
        pltpu.VMEM((CH,), jnp.int32),
        pltpu.VMEM((CH + 16,), jnp.int32),
        pltpu.VMEM((CH + 16,), jnp.float32),
        pltpu.VMEM((CH, H), jnp.float32),
        pltpu.VMEM((H,), jnp.float32),
        pltpu.VMEM((112,), jnp.int32),
        pltpu.SemaphoreType.DMA,
    ],
)
def _sc_scatter(h_hbm, ssrc_hbm, sdst_hbm, snorm_hbm, se_hbm,
                out_hbm, headp_hbm,
                idx_v, dst_v, nrm_v, rowbuf, accbuf, se_v, sem):
    cid = lax.axis_index("c")
    sid = lax.axis_index("s")
    wid = sid * 2 + cid

    pltpu.sync_copy(se_hbm, se_v)
    start = se_v[pl.ds(wid, 16)][0]
    end = se_v[pl.ds(32 + wid, 16)][0]
    ismid = se_v[pl.ds(64 + wid, 16)][0]

    z16 = jnp.zeros((16,), jnp.float32)
    for k in range(16):
        accbuf[pl.ds(16 * k, 16)] = z16
    pltpu.sync_copy(accbuf, headp_hbm.at[wid])

    @pl.when(end > start)
    def _():
        abase = jnp.bitwise_and(start, jnp.int32(-16))
        nch = (end - abase + (CH - 1)) // CH

        def flush(cur, first, acc):
            for k in range(16):
                accbuf[pl.ds(16 * k, 16)] = acc[k]

            def to_head():
                pltpu.sync_copy(accbuf, headp_hbm.at[wid])

            def to_out():
                pltpu.sync_copy(accbuf, out_hbm.at[cur])

            lax.cond(jnp.logical_and(first == 1, ismid == 1), to_head, to_out)

        def chunk_body(ci, carry):
            cbase = pl.multiple_of(abase + ci * CH, 16)
            pltpu.sync_copy(ssrc_hbm.at[pl.ds(cbase, CH)], idx_v)
            pltpu.sync_copy(sdst_hbm.at[pl.ds(cbase, CH)], dst_v.at[pl.ds(0, CH)])
            pltpu.sync_copy(snorm_hbm.at[pl.ds(cbase, CH)], nrm_v.at[pl.ds(0, CH)])
            pltpu.async_copy(h_hbm.at[idx_v], rowbuf, sem).wait()
            lo = jnp.maximum(start - cbase, 0)
            hi = jnp.minimum(end - cbase, CH)

            def edge_body(j, ec):
                cur = ec[0]
                first = ec[1]
                acc = ec[2]
                d = dst_v[pl.ds(j, 16)][0]
                nv = nrm_v[pl.ds(j, 16)][0]
                changed = jnp.logical_and(d != cur, cur >= 0)
                lax.cond(changed, lambda: flush(cur, first, acc), lambda: None)
                first = jnp.where(changed, 0, first)
                acc = tuple(jnp.where(changed, z16, acc[k]) for k in range(16))
                acc = tuple(acc[k] + rowbuf[j, pl.ds(16 * k, 16)] * nv
                            for k in range(16))
                return (d, first, acc)

            return lax.fori_loop(lo, hi, edge_body, carry)

        carry0 = (jnp.int32(-1), jnp.int32(1), tuple(z16 for _ in range(16)))
        cur, first, acc = lax.fori_loop(0, nch, chunk_body, carry0)
        flush(cur, first, acc)


@functools.partial(
    pl.kernel,
    out_type=(
        jax.ShapeDtypeStruct((LPAD // 128, 128), jnp.int32),
        jax.ShapeDtypeStruct((LPAD // 128, 128), jnp.int32),
        jax.ShapeDtypeStruct((LPAD // 128, 128), jnp.float32),
    ),
    mesh=_mesh,
    scratch_types=[
        pltpu.VMEM((8, 128), jnp.int32),
        pltpu.VMEM((8, 128), jnp.int32),
        pltpu.VMEM((8, 128), jnp.int32),
        pltpu.VMEM((8, 128), jnp.float32),
        pltpu.VMEM((8, 128), jnp.float32),
        pltpu.SemaphoreType.DMA,
    ],
)
def _sc_setup(scat_hbm, dcat_hbm, perm_hbm, dis_hbm,
              ssrc_hbm, sdst_hbm, snorm_hbm,
              pv, sv, dv, av, bv, sem):
    cid = lax.axis_index("c")
    sid = lax.axis_index("s")
    wid = sid * 2 + cid
    base = wid * SPER

    row0 = wid * (SPER // 128)

    def chunk(ci, _):
        r0 = row0 + ci * 8
        pltpu.sync_copy(perm_hbm.at[pl.ds(r0, 8)], pv)
        hs = [pltpu.async_copy(scat_hbm.at[pv.at[r]], sv.at[r], sem)
              for r in range(8)]
        hs += [pltpu.async_copy(dcat_hbm.at[pv.at[r]], dv.at[r], sem)
               for r in range(8)]
        for hh in hs:
            hh.wait()
        hs = [pltpu.async_copy(dis_hbm.at[sv.at[r]], av.at[r], sem)
              for r in range(8)]
        hs += [pltpu.async_copy(dis_hbm.at[dv.at[r]], bv.at[r], sem)
               for r in range(8)]
        for hh in hs:
            hh.wait()
        for r in range(8):
            for k in range(8):
                av[r, pl.ds(16 * k, 16)] = (av[r, pl.ds(16 * k, 16)] *
                                            bv[r, pl.ds(16 * k, 16)])
        pltpu.sync_copy(sv, ssrc_hbm.at[pl.ds(r0, 8)])
        pltpu.sync_copy(dv, sdst_hbm.at[pl.ds(r0, 8)])
        pltpu.sync_copy(av, snorm_hbm.at[pl.ds(r0, 8)])
        return 0

    lax.fori_loop(0, SPER // SCH, chunk, 0)


def _aggregate(h, ssrc_p, sdst_p, snorm_p, se, headrow):
    out, headp = _sc_scatter(h, ssrc_p, sdst_p, snorm_p, se)
    return out.at[headrow].add(headp)


def _bn(x, g, b):
    m = jnp.mean(x, axis=0)
    v = jnp.var(x, axis=0)
    return (x - m) * jax.lax.rsqrt(v + EPS) * g + b


def _pools(x, batch):
    cnt = jax.ops.segment_sum(jnp.ones((x.shape[0],), x.dtype), batch, num_segments=G)
    sm = jax.ops.segment_sum(x, batch, num_segments=G)
    mean = sm / jnp.maximum(cnt, 1.0)[:, None]
    mx = jax.ops.segment_max(x, batch, num_segments=G)
    mx = jnp.where(cnt[:, None] > 0, mx, 0.0)
    return jnp.concatenate([mean, mx], axis=1)


def kernel(x, edge_index, batch, W1, b1, W2, b2, W3, b3, W4, b4, g1, be1, g2, be2, g3, be3, g4, be4, Ws1, bs1, Ws2, bs2, Ha1, ha1, Hb1, hb1, Ha2, ha2, Hb2, hb2, Ha3, ha3, Hb3, hb3):
    src = edge_index[0]
    dst = edge_index[1]
    loop = jnp.arange(N, dtype=src.dtype)
    scat = jnp.concatenate([src, loop])
    dcat = jnp.concatenate([dst, loop])
    deg = jnp.zeros((N,), jnp.float32).at[dcat].add(1.0)
    dis = jax.lax.rsqrt(deg)

    perm = jnp.argsort(dcat, stable=True)
    perm_p = jnp.concatenate(
        [perm.astype(jnp.int32), jnp.zeros((LPAD - ECAT,), jnp.int32)])
    ssrc2, sdst2, snorm2 = _sc_setup(
        scat, dcat, perm_p.reshape(LPAD // 128, 128), dis)
    ssrc_p = ssrc2.reshape(LPAD)
    sdst_p = sdst2.reshape(LPAD)
    snorm_p = snorm2.reshape(LPAD)
    sdst = sdst_p[:ECAT]

    starts, ends = [], []
    for (s0, e0) in _R16:
        m = (s0 + e0) // 2
        cutv = jnp.searchsorted(sdst, sdst[m], side='left').astype(jnp.int32)
        cut = jnp.clip(cutv, s0, e0)
        starts += [jnp.int32(s0), cut]
        ends += [cut, jnp.int32(e0)]
    starts = jnp.stack(starts)
    ends = jnp.stack(ends)
    prev = jnp.where(starts > 0, starts - 1, 0)
    is_mid = (starts > 0) & (sdst_p[starts] == sdst_p[prev]) & (ends > starts)
    se = jnp.concatenate([starts, ends, is_mid.astype(jnp.int32),
                          jnp.zeros((16,), jnp.int32)])
    headrow = jnp.where(is_mid, sdst_p[starts], -1)

    def gcn(h_in, W, b):
        h = h_in @ W
        out = _aggregate(h, ssrc_p, sdst_p, snorm_p, se, headrow)
        return out + b

    h = jax.nn.relu(_bn(gcn(x, W1, b1), g1, be1))
    h = jax.nn.relu(_bn(gcn(h, W2, b2), g2, be2))
    h = jax.nn.relu(_bn(gcn(h, W3, b3), g3, be3))
    h = jax.nn.relu(_bn(gcn(h, W4, b4), g4, be4))

    p = _pools(h, batch)
    s = jax.nn.relu(p @ Ws1 + bs1)
    s = jax.nn.relu(s @ Ws2 + bs2)
    outs = []
    for Ha, ha, Hb, hb in ((Ha1, ha1, Hb1, hb1), (Ha2, ha2, Hb2, hb2), (Ha3, ha3, Hb3, hb3)):
        t = jax.nn.relu(s @ Ha + ha)
        outs.append((t @ Hb + hb).squeeze(-1))
    return tuple(outs)


# speedup vs baseline: 5.8281x; 1.2912x over previous
import functools

import jax
import jax.numpy as jnp
from jax import lax
from jax.experimental import pallas as pl
from jax.experimental.pallas import tpu as pltpu
from jax.experimental.pallas import tpu_sc as plsc

N = 50000
E = 800000
H = 256
G = 128
EPS = 1e-5
ECAT = E + N
WIN = 112
CH = 128
SCH = 1024
SPER = 26 * SCH
LPAD = 32 * SPER


def _tile_ranges(e_tot):
    n_win = -(-e_tot // WIN)
    base, extra = divmod(n_win, 16)
    ends, pos = [], 0
    for t in range(16):
        pos += (base + (1 if t < extra else 0)) * WIN
        ends.append(min(pos, e_tot))
    return list(zip([0] + ends[:-1], ends))


_R16 = _tile_ranges(ECAT)

_mesh = plsc.VectorSubcoreMesh(core_axis_name="c", subcore_axis_name="s")


@functools.partial(
    pl.kernel,
    out_type=(
        jax.ShapeDtypeStruct((N, H), jnp.float32),
        jax.ShapeDtypeStruct((32, H), jnp.float32),
    ),
    mesh=_mesh,
    scratch_types=[
        pltpu.VMEM((CH,), jnp.int32),
        pltpu.VMEM((CH,), jnp.int32),
        pltpu.VMEM((2 * (CH + 16),), jnp.int32),
        pltpu.VMEM((2 * (CH + 16),), jnp.float32),
        pltpu.VMEM((CH, H), jnp.float32),
        pltpu.VMEM((CH, H), jnp.float32),
        pltpu.VMEM((H,), jnp.float32),
        pltpu.VMEM((H,), jnp.float32),
        pltpu.VMEM((112,), jnp.int32),
        pltpu.SemaphoreType.DMA,
        pltpu.SemaphoreType.DMA,
        pltpu.SemaphoreType.DMA,
    ],
)
def _sc_scatter(h_hbm, ssrc_hbm, sdst_hbm, snorm_hbm, se_hbm,
                out_hbm, headp_hbm,
                idx0, idx1, dst_v, nrm_v, row0, row1, fl0, fl1, se_v,
                isem, gsem, fsem):
    cid = lax.axis_index("c")
    sid = lax.axis_index("s")
    wid = sid * 2 + cid

    pltpu.sync_copy(se_hbm, se_v)
    start = se_v[pl.ds(wid, 16)][0]
    end = se_v[pl.ds(32 + wid, 16)][0]
    ismid = se_v[pl.ds(64 + wid, 16)][0]

    z16 = jnp.zeros((16,), jnp.float32)
    for k in range(16):
        fl0[pl.ds(16 * k, 16)] = z16
    pltpu.sync_copy(fl0, headp_hbm.at[wid])

    @pl.when(end > start)
    def _():
        abase = jnp.bitwise_and(start, jnp.int32(-16))
        nch = (end - abase + (CH - 1)) // CH

        def cb(ci):
            return pl.multiple_of(abase + ci * CH, 16)

        def stage_idx(ci, b):
            cbase = cb(ci)
            pltpu.async_copy(ssrc_hbm.at[pl.ds(cbase, CH)],
                             idx0 if b == 0 else idx1, isem)
            pltpu.async_copy(sdst_hbm.at[pl.ds(cbase, CH)],
                             dst_v.at[pl.ds(b * (CH + 16), CH)], isem)
            pltpu.async_copy(snorm_hbm.at[pl.ds(cbase, CH)],
                             nrm_v.at[pl.ds(b * (CH + 16), CH)], isem)

        def wait_idx(ci, b):
            cbase = cb(ci)
            pltpu.make_async_copy(ssrc_hbm.at[pl.ds(cbase, CH)],
                                  idx0 if b == 0 else idx1, isem).wait()
            pltpu.make_async_copy(sdst_hbm.at[pl.ds(cbase, CH)],
                                  dst_v.at[pl.ds(b * (CH + 16), CH)], isem).wait()
            pltpu.make_async_copy(snorm_hbm.at[pl.ds(cbase, CH)],
                                  nrm_v.at[pl.ds(b * (CH + 16), CH)], isem).wait()

        def fire_gather(b):
            pltpu.async_copy(h_hbm.at[idx0 if b == 0 else idx1],
                             row0 if b == 0 else row1, gsem)

        def wait_gather(b):
            pltpu.make_async_copy(h_hbm.at[idx0 if b == 0 else idx1],
                                  row0 if b == 0 else row1, gsem).wait()

        def drain_flush(sl):
            def d0():
                pltpu.make_async_copy(out_hbm.at[0], fl0, fsem).wait()

            def d1():
                pltpu.make_async_copy(out_hbm.at[0], fl1, fsem).wait()

            lax.cond(sl == 0, d0, d1)

        def flush(cur, first, acc, fcnt, pend0, pend1):
            sl = lax.rem(fcnt, 2)
            pend = jnp.where(sl == 0, pend0, pend1)
            lax.cond(pend == 1, lambda: drain_flush(sl), lambda: None)

            def emit(buf):
                for k in range(16):
                    buf[pl.ds(16 * k, 16)] = acc[k]

                def to_head():
                    pltpu.async_copy(buf, headp_hbm.at[wid], fsem)

                def to_out():
                    pltpu.async_copy(buf, out_hbm.at[cur], fsem)

                lax.cond(jnp.logical_and(first == 1, ismid == 1),
                         to_head, to_out)

            lax.cond(sl == 0, lambda: emit(fl0), lambda: emit(fl1))
            pend0 = jnp.where(sl == 0, 1, pend0)
            pend1 = jnp.where(sl == 1, 1, pend1)
            return fcnt + 1, pend0, pend1

        def scan_chunk(ci, b, carry):
            cbase = cb(ci)
            lo = jnp.clip(start - cbase, 0, CH)
            hi = jnp.clip(end - cbase, 0, CH)

            def edge_body(j, ec):
                cur, first, fcnt, pend0, pend1 = ec[0], ec[1], ec[2], ec[3], ec[4]
                acc = ec[5]
                d = dst_v[pl.ds(b * (CH + 16) + j, 16)][0]
                nv = nrm_v[pl.ds(b * (CH + 16) + j, 16)][0]
                changed = jnp.logical_and(d != cur, cur >= 0)
                fcnt, pend0, pend1 = lax.cond(
                    changed,
                    lambda: flush(cur, first, acc, fcnt, pend0, pend1),
                    lambda: (fcnt, pend0, pend1))
                first = jnp.where(changed, 0, first)
                acc = tuple(jnp.where(changed, z16, acc[k]) for k in range(16))
                rb = row0 if b == 0 else row1
                acc = tuple(acc[k] + rb[j, pl.ds(16 * k, 16)] * nv
                            for k in range(16))
                return (d, first, fcnt, pend0, pend1, acc)

            return lax.fori_loop(lo, hi, edge_body, carry)

        stage_idx(0, 0)
        wait_idx(0, 0)
        fire_gather(0)
        lax.cond(nch > 1, lambda: stage_idx(1, 1), lambda: None)

        def step(c, b, carry):
            def prep():
                wait_idx(c + 1, 1 - b)
                fire_gather(1 - b)

            lax.cond(c + 1 < nch, prep, lambda: None)
            lax.cond(c < nch, lambda: wait_gather(b), lambda: None)
            carry = scan_chunk(c, b, carry)
            lax.cond(c + 2 < nch, lambda: stage_idx(c + 2, b), lambda: None)
            return carry

        carry0 = (jnp.int32(-1), jnp.int32(1), jnp.int32(0), jnp.int32(0),
                  jnp.int32(0), tuple(z16 for _ in range(16)))

        def pair_body(p, carry):
            carry = step(2 * p, 0, carry)
            carry = step(2 * p + 1, 1, carry)
            return carry

        carry = lax.fori_loop(0, (nch + 1) // 2, pair_body, carry0)
        cur, first, fcnt, pend0, pend1, acc = carry
        fcnt, pend0, pend1 = flush(cur, first, acc, fcnt, pend0, pend1)
        lax.cond(pend0 == 1, lambda: drain_flush(jnp.int32(0)), lambda: None)
        lax.cond(pend1 == 1, lambda: drain_flush(jnp.int32(1)), lambda: None)


@functools.partial(
    pl.kernel,
    out_type=(
        jax.ShapeDtypeStruct((LPAD // 128, 128), jnp.int32),
        jax.ShapeDtypeStruct((LPAD // 128, 128), jnp.int32),
        jax.ShapeDtypeStruct((LPAD // 128, 128), jnp.float32),
    ),
    mesh=_mesh,
    scratch_types=[
        pltpu.VMEM((8, 128), jnp.int32),
        pltpu.VMEM((8, 128), jnp.int32),
        pltpu.VMEM((8, 128), jnp.int32),
        pltpu.VMEM((8, 128), jnp.float32),
        pltpu.VMEM((8, 128), jnp.float32),
        pltpu.SemaphoreType.DMA,
    ],
)
def _sc_setup(scat_hbm, dcat_hbm, perm_hbm, dis_hbm,
              ssrc_hbm, sdst_hbm, snorm_hbm,
              pv, sv, dv, av, bv, sem):
    cid = lax.axis_index("c")
    sid = lax.axis_index("s")
    wid = sid * 2 + cid
    base = wid * SPER

    row0 = wid * (SPER // 128)

    def chunk(ci, _):
        r0 = row0 + ci * 8
        pltpu.sync_copy(perm_hbm.at[pl.ds(r0, 8)], pv)
        hs = [pltpu.async_copy(scat_hbm.at[pv.at[r]], sv.at[r], sem)
              for r in range(8)]
        hs += [pltpu.async_copy(dcat_hbm.at[pv.at[r]], dv.at[r], sem)
               for r in range(8)]
        for hh in hs:
            hh.wait()
        hs = [pltpu.async_copy(dis_hbm.at[sv.at[r]], av.at[r], sem)
              for r in range(8)]
        hs += [pltpu.async_copy(dis_hbm.at[dv.at[r]], bv.at[r], sem)
               for r in range(8)]
        for hh in hs:
            hh.wait()
        for r in range(8):
            for k in range(8):
                av[r, pl.ds(16 * k, 16)] = (av[r, pl.ds(16 * k, 16)] *
                                            bv[r, pl.ds(16 * k, 16)])
        pltpu.sync_copy(sv, ssrc_hbm.at[pl.ds(r0, 8)])
        pltpu.sync_copy(dv, sdst_hbm.at[pl.ds(r0, 8)])
        pltpu.sync_copy(av, snorm_hbm.at[pl.ds(r0, 8)])
        return 0

    lax.fori_loop(0, SPER // SCH, chunk, 0)


def _aggregate(h, ssrc_p, sdst_p, snorm_p, se, headrow):
    out, headp = _sc_scatter(h, ssrc_p, sdst_p, snorm_p, se)
    return out.at[headrow].add(headp)


def _bn(x, g, b):
    m = jnp.mean(x, axis=0)
    v = jnp.var(x, axis=0)
    return (x - m) * jax.lax.rsqrt(v + EPS) * g + b


def _pools(x, batch):
    cnt = jax.ops.segment_sum(jnp.ones((x.shape[0],), x.dtype), batch, num_segments=G)
    sm = jax.ops.segment_sum(x, batch, num_segments=G)
    mean = sm / jnp.maximum(cnt, 1.0)[:, None]
    mx = jax.ops.segment_max(x, batch, num_segments=G)
    mx = jnp.where(cnt[:, None] > 0, mx, 0.0)
    return jnp.concatenate([mean, mx], axis=1)


def kernel(x, edge_index, batch, W1, b1, W2, b2, W3, b3, W4, b4, g1, be1, g2, be2, g3, be3, g4, be4, Ws1, bs1, Ws2, bs2, Ha1, ha1, Hb1, hb1, Ha2, ha2, Hb2, hb2, Ha3, ha3, Hb3, hb3):
    src = edge_index[0]
    dst = edge_index[1]
    loop = jnp.arange(N, dtype=src.dtype)
    scat = jnp.concatenate([src, loop])
    dcat = jnp.concatenate([dst, loop])
    deg = jnp.zeros((N,), jnp.float32).at[dcat].add(1.0)
    dis = jax.lax.rsqrt(deg)

    perm = jnp.argsort(dcat, stable=True)
    perm_p = jnp.concatenate(
        [perm.astype(jnp.int32), jnp.zeros((LPAD - ECAT,), jnp.int32)])
    ssrc2, sdst2, snorm2 = _sc_setup(
        scat, dcat, perm_p.reshape(LPAD // 128, 128), dis)
    ssrc_p = ssrc2.reshape(LPAD)
    sdst_p = sdst2.reshape(LPAD)
    snorm_p = snorm2.reshape(LPAD)
    sdst = sdst_p[:ECAT]

    starts, ends = [], []
    for (s0, e0) in _R16:
        m = (s0 + e0) // 2
        cutv = jnp.searchsorted(sdst, sdst[m], side='left').astype(jnp.int32)
        cut = jnp.clip(cutv, s0, e0)
        starts += [jnp.int32(s0), cut]
        ends += [cut, jnp.int32(e0)]
    starts = jnp.stack(starts)
    ends = jnp.stack(ends)
    prev = jnp.where(starts > 0, starts - 1, 0)
    is_mid = (starts > 0) & (sdst_p[starts] == sdst_p[prev]) & (ends > starts)
    se = jnp.concatenate([starts, ends, is_mid.astype(jnp.int32),
                          jnp.zeros((16,), jnp.int32)])
    headrow = jnp.where(is_mid, sdst_p[starts], -1)

    def gcn(h_in, W, b):
        h = h_in @ W
        out = _aggregate(h, ssrc_p, sdst_p, snorm_p, se, headrow)
        return out + b

    h = jax.nn.relu(_bn(gcn(x, W1, b1), g1, be1))
    h = jax.nn.relu(_bn(gcn(h, W2, b2), g2, be2))
    h = jax.nn.relu(_bn(gcn(h, W3, b3), g3, be3))
    h = jax.nn.relu(_bn(gcn(h, W4, b4), g4, be4))

    p = _pools(h, batch)
    s = jax.nn.relu(p @ Ws1 + bs1)
    s = jax.nn.relu(s @ Ws2 + bs2)
    outs = []
    for Ha, ha, Hb, hb in ((Ha1, ha1, Hb1, hb1), (Ha2, ha2, Hb2, hb2), (Ha3, ha3, Hb3, hb3)):
        t = jax.nn.relu(s @ Ha + ha)
        outs.append((t @ Hb + hb).squeeze(-1))
    return tuple(outs)
